# baseline (speedup 1.0000x reference)
import functools

import jax
import jax.numpy as jnp
from jax import lax
from jax.experimental import pallas as pl
from jax.experimental.pallas import tpu as pltpu
from jax.experimental.pallas import tpu_sc as plsc

B = 16384
EMB = 32
L = 16
NC = 2
NS = 16
NW = NC * NS
BPW = B // NW
CHUNK = 128
NCHUNK = BPW // CHUNK


def _permute(v, p):
    return lax.gather(
        v, p[:, None],
        lax.GatherDimensionNumbers(offset_dims=(), collapsed_slice_dims=(0,),
                                   start_index_map=(0,)),
        slice_sizes=(1,),
        mode=lax.GatherScatterMode.PROMISE_IN_BOUNDS)


def _sc_body(uid_hbm, iid_hbm, uemb_hbm, iemb_hbm, out_hbm,
             uidx_v, iidx_v, urows_v, irows_v, out_v, sem):
    wid = lax.axis_index("s") * NC + lax.axis_index("c")
    base = wid * BPW

    pltpu.sync_copy(uid_hbm.at[pl.ds(base, BPW)], uidx_v)
    pltpu.sync_copy(iid_hbm.at[pl.ds(base, BPW)], iidx_v)

    copies = []
    for j in range(NCHUNK):
        sl = pl.ds(j * CHUNK, CHUNK)
        copies.append(pltpu.async_copy(uemb_hbm.at[uidx_v.at[sl]],
                                       urows_v.at[sl], sem))
        copies.append(pltpu.async_copy(iemb_hbm.at[iidx_v.at[sl]],
                                       irows_v.at[sl], sem))
    for c in copies:
        c.wait()

    lanes = lax.iota(jnp.int32, L)
    perms = [jnp.arange(L, dtype=jnp.int32) ^ (1 << k) for k in range(4)]

    def grpfn(g, carry):
        b0 = g * L
        acc = jnp.zeros((L,), jnp.float32)
        for r in range(L):
            b = b0 + r
            u0 = urows_v[b, pl.ds(0, L)]
            u1 = urows_v[b, pl.ds(L, L)]
            i0 = irows_v[b, pl.ds(0, L)]
            i1 = irows_v[b, pl.ds(L, L)]
            v = u0 * i0 + u1 * i1
            for p in perms:
                v = v + _permute(v, p)
            acc = jnp.where(lanes == r, v, acc)
        out_v[pl.ds(b0, L)] = acc
        return carry

    lax.fori_loop(0, BPW // L, grpfn, 0, unroll=1)

    pltpu.sync_copy(out_v, out_hbm.at[pl.ds(base, BPW)])


@jax.jit
def _run(user_ids, item_ids, user_emb, item_emb):
    mesh = plsc.VectorSubcoreMesh(core_axis_name="c", subcore_axis_name="s")
    f = functools.partial(
        pl.kernel,
        out_type=jax.ShapeDtypeStruct((B,), jnp.float32),
        mesh=mesh,
        compiler_params=pltpu.CompilerParams(use_tc_tiling_on_sc=False),
        scratch_types=[
            pltpu.VMEM((BPW,), jnp.int32),
            pltpu.VMEM((BPW,), jnp.int32),
            pltpu.VMEM((BPW, EMB), jnp.float32),
            pltpu.VMEM((BPW, EMB), jnp.float32),
            pltpu.VMEM((BPW,), jnp.float32),
            pltpu.SemaphoreType.DMA,
        ],
    )(_sc_body)
    return f(user_ids, item_ids, user_emb, item_emb)


def kernel(user_ids, item_ids, user_emb, item_emb):
    return _run(user_ids.astype(jnp.int32), item_ids.astype(jnp.int32),
                user_emb, item_emb)
